# baseline (speedup 1.0000x reference)
import jax
import jax.numpy as jnp
from jax.experimental import pallas as pl
from jax.experimental.pallas import tpu as pltpu

N_CLUSTERS = 128
MAX_ITER = 20
TOL = 1e-4
MAX_NEIGHBORS = 15
N_POINTS = 16384
PAD_D = 8

_NEG_INF = float("-inf")


def _kmeans_kernel(xp_ref, xt_ref, xtb_ref, c_ref, cents_ref, closest_ref,
                   v_ref):
    xp = xp_ref[0]
    xt = xt_ref[0]
    xtb = xtb_ref[0]
    c0 = c_ref[0]

    xsq = jnp.sum(xt * xt, axis=0, keepdims=True)
    iota_cl = jax.lax.broadcasted_iota(jnp.int32, (N_CLUSTERS, N_POINTS), 0)
    iota_pt = jax.lax.broadcasted_iota(jnp.int32, (N_CLUSTERS, N_POINTS), 1)

    def sim_of(c):
        ct = jnp.transpose(c)
        csq = jnp.transpose(jnp.sum(ct * ct, axis=0, keepdims=True))
        d2 = jnp.dot((2.0 * c).astype(jnp.bfloat16), xtb,
                     preferred_element_type=jnp.float32)
        return (d2 - xsq) - csq

    col_d = jax.lax.broadcasted_iota(jnp.int32, (N_CLUSTERS, PAD_D), 1)

    def body(_, carry):
        closest, cents, cntc, err = carry
        active = err > TOL
        sim = sim_of(cents)
        m = jnp.max(sim, axis=0, keepdims=True)
        new_closest = jnp.min(
            jnp.where(sim == m, iota_cl, N_CLUSTERS), axis=0, keepdims=True)
        mask = (iota_cl == new_closest).astype(jnp.bfloat16)
        s = jnp.dot(mask, xp, preferred_element_type=jnp.float32)
        cnt = s[:, 3:4]
        cg = jnp.where(col_d >= 3, 0.0, s / (cnt + 1e-8))
        new_err = jnp.sum((cg - cents) ** 2)
        return (jnp.where(active, new_closest, closest),
                jnp.where(active, cg, cents),
                jnp.where(active, cnt, cntc),
                jnp.where(active, new_err, err))

    init = (jnp.zeros((1, N_POINTS), jnp.int32), c0,
            jnp.zeros((N_CLUSTERS, 1), jnp.float32), jnp.float32(jnp.inf))
    closest, cents, cnt, _ = jax.lax.fori_loop(0, MAX_ITER, body, init)

    cents_ref[0] = cents
    closest_ref[0] = closest

    sim = sim_of(cents)
    in_cluster = iota_cl == closest
    work = jnp.where(in_cluster, sim, _NEG_INF)
    colmax = jnp.max(work, axis=1, keepdims=True)
    e = jnp.exp(work - colmax)

    denom = jnp.zeros((N_CLUSTERS, 1), jnp.float32)
    for k in range(MAX_NEIGHBORS):
        mk = colmax if k == 0 else jnp.max(work, axis=1, keepdims=True)
        rk = jnp.min(jnp.where(work == mk, iota_pt, N_POINTS),
                     axis=1, keepdims=True)
        denom = denom + jnp.exp(mk - colmax)
        work = jnp.where(iota_pt == rk, _NEG_INF, work)

    selected = jnp.logical_and(in_cluster, work == _NEG_INF)
    v_ref[0] = jnp.sum(jnp.where(selected, e / denom, 0.0),
                       axis=0, keepdims=True)


def _feat_kernel(cl_ref, v_ref, f_ref, o_ref):
    t = pl.program_id(1)
    kb = f_ref.shape[2]
    iota_cl = jax.lax.broadcasted_iota(jnp.int32, (N_CLUSTERS, kb), 0)
    wt = jnp.where(iota_cl == cl_ref[0], v_ref[0], 0.0)

    @pl.when(t == 0)
    def _():
        o_ref[...] = jnp.zeros_like(o_ref)

    o_ref[0] += jax.lax.dot_general(
        f_ref[0], wt, (((1,), (1,)), ((), ())),
        preferred_element_type=jnp.float32,
        precision=jax.lax.Precision.HIGHEST)


@jax.jit
def kernel(points, features, centroids):
    B = points.shape[0]
    F = features.shape[1]
    xp0 = jnp.pad(points, ((0, 0), (0, 0), (0, PAD_D - 3)))
    xt = jnp.transpose(xp0, (0, 2, 1))
    xtb = xt.astype(jnp.bfloat16)
    xp = jnp.concatenate(
        [points, jnp.ones((B, N_POINTS, 1), jnp.float32),
         jnp.zeros((B, N_POINTS, PAD_D - 4), jnp.float32)],
        axis=2).astype(jnp.bfloat16)
    cp = jnp.pad(centroids, ((0, 0), (0, 0), (0, PAD_D - 3)))

    cents, closest, v = pl.pallas_call(
        _kmeans_kernel,
        grid=(B,),
        in_specs=[
            pl.BlockSpec((1, N_POINTS, PAD_D), lambda b: (b, 0, 0)),
            pl.BlockSpec((1, PAD_D, N_POINTS), lambda b: (b, 0, 0)),
            pl.BlockSpec((1, PAD_D, N_POINTS), lambda b: (b, 0, 0)),
            pl.BlockSpec((1, N_CLUSTERS, PAD_D), lambda b: (b, 0, 0)),
        ],
        out_specs=[
            pl.BlockSpec((1, N_CLUSTERS, PAD_D), lambda b: (b, 0, 0)),
            pl.BlockSpec((1, 1, N_POINTS), lambda b: (b, 0, 0)),
            pl.BlockSpec((1, 1, N_POINTS), lambda b: (b, 0, 0)),
        ],
        out_shape=[
            jax.ShapeDtypeStruct((B, N_CLUSTERS, PAD_D), jnp.float32),
            jax.ShapeDtypeStruct((B, 1, N_POINTS), jnp.int32),
            jax.ShapeDtypeStruct((B, 1, N_POINTS), jnp.float32),
        ],
        compiler_params=pltpu.CompilerParams(
            dimension_semantics=("parallel",)),
    )(xp, xt, xtb, cp)

    KT = 4
    KB = N_POINTS // KT
    cf = pl.pallas_call(
        _feat_kernel,
        grid=(B, KT),
        in_specs=[
            pl.BlockSpec((1, 1, KB), lambda b, t: (b, 0, t)),
            pl.BlockSpec((1, 1, KB), lambda b, t: (b, 0, t)),
            pl.BlockSpec((1, F, KB), lambda b, t: (b, 0, t)),
        ],
        out_specs=pl.BlockSpec((1, F, N_CLUSTERS), lambda b, t: (b, 0, 0)),
        out_shape=jax.ShapeDtypeStruct((B, F, N_CLUSTERS), jnp.float32),
        compiler_params=pltpu.CompilerParams(
            dimension_semantics=("parallel", "arbitrary")),
    )(closest, v, features)

    cc = cents[:, :, :3]
    return cc, cf, closest[:, 0, :]


# speedup vs baseline: 1.0554x; 1.0554x over previous
import jax
import jax.numpy as jnp
from jax.experimental import pallas as pl
from jax.experimental.pallas import tpu as pltpu

N_CLUSTERS = 128
MAX_ITER = 20
TOL = 1e-4
MAX_NEIGHBORS = 15
N_POINTS = 16384
PAD_D = 8

_NEG_INF = float("-inf")


def _kmeans_kernel(xp_ref, xt_ref, c_ref, cents_ref, closest_ref, v_ref):
    xp = xp_ref[0]
    xt = xt_ref[0]
    c0 = c_ref[0]

    xsq = jnp.sum(xt * xt, axis=0, keepdims=True)
    iota_cl = jax.lax.broadcasted_iota(jnp.int32, (N_CLUSTERS, N_POINTS), 0)
    iota_pt = jax.lax.broadcasted_iota(jnp.int32, (N_CLUSTERS, N_POINTS), 1)

    def sim_of(c):
        ct = jnp.transpose(c)
        csq = jnp.transpose(jnp.sum(ct * ct, axis=0, keepdims=True))
        d2 = jnp.dot(2.0 * c, xt,
                     preferred_element_type=jnp.float32)
        return (d2 - xsq) - csq

    col_d = jax.lax.broadcasted_iota(jnp.int32, (N_CLUSTERS, PAD_D), 1)

    def body(_, carry):
        closest, cents, cntc, err = carry
        active = err > TOL
        sim = sim_of(cents)
        m = jnp.max(sim, axis=0, keepdims=True)
        new_closest = jnp.min(
            jnp.where(sim == m, iota_cl, N_CLUSTERS), axis=0, keepdims=True)
        mask = (iota_cl == new_closest).astype(jnp.bfloat16)
        s = jnp.dot(mask, xp, preferred_element_type=jnp.float32)
        cnt = s[:, 3:4]
        cg = jnp.where(col_d >= 3, 0.0, s / (cnt + 1e-8))
        new_err = jnp.sum((cg - cents) ** 2)
        return (jnp.where(active, new_closest, closest),
                jnp.where(active, cg, cents),
                jnp.where(active, cnt, cntc),
                jnp.where(active, new_err, err))

    init = (jnp.zeros((1, N_POINTS), jnp.int32), c0,
            jnp.zeros((N_CLUSTERS, 1), jnp.float32), jnp.float32(jnp.inf))
    closest, cents, cnt, _ = jax.lax.fori_loop(0, MAX_ITER, body, init)

    cents_ref[0] = cents
    closest_ref[0] = closest

    sim = sim_of(cents)
    in_cluster = iota_cl == closest
    work = jnp.where(in_cluster, sim, _NEG_INF)
    colmax = jnp.max(work, axis=1, keepdims=True)
    e = jnp.exp(work - colmax)

    denom = jnp.zeros((N_CLUSTERS, 1), jnp.float32)
    for k in range(MAX_NEIGHBORS):
        mk = colmax if k == 0 else jnp.max(work, axis=1, keepdims=True)
        rk = jnp.min(jnp.where(work == mk, iota_pt, N_POINTS),
                     axis=1, keepdims=True)
        denom = denom + jnp.exp(mk - colmax)
        work = jnp.where(iota_pt == rk, _NEG_INF, work)

    selected = jnp.logical_and(in_cluster, work == _NEG_INF)
    v_ref[0] = jnp.sum(jnp.where(selected, e / denom, 0.0),
                       axis=0, keepdims=True)


def _feat_kernel(cl_ref, v_ref, f_ref, o_ref):
    t = pl.program_id(1)
    kb = f_ref.shape[2]
    iota_cl = jax.lax.broadcasted_iota(jnp.int32, (N_CLUSTERS, kb), 0)
    wt = jnp.where(iota_cl == cl_ref[0], v_ref[0], 0.0)

    @pl.when(t == 0)
    def _():
        o_ref[...] = jnp.zeros_like(o_ref)

    o_ref[0] += jax.lax.dot_general(
        f_ref[0], wt, (((1,), (1,)), ((), ())),
        preferred_element_type=jnp.float32,
        precision=jax.lax.Precision.HIGHEST)


@jax.jit
def kernel(points, features, centroids):
    B = points.shape[0]
    F = features.shape[1]
    xp0 = jnp.pad(points, ((0, 0), (0, 0), (0, PAD_D - 3)))
    xt = jnp.transpose(xp0, (0, 2, 1))
    xp = jnp.concatenate(
        [points, jnp.ones((B, N_POINTS, 1), jnp.float32),
         jnp.zeros((B, N_POINTS, PAD_D - 4), jnp.float32)],
        axis=2).astype(jnp.bfloat16)
    cp = jnp.pad(centroids, ((0, 0), (0, 0), (0, PAD_D - 3)))

    cents, closest, v = pl.pallas_call(
        _kmeans_kernel,
        grid=(B,),
        in_specs=[
            pl.BlockSpec((1, N_POINTS, PAD_D), lambda b: (b, 0, 0)),
            pl.BlockSpec((1, PAD_D, N_POINTS), lambda b: (b, 0, 0)),
            pl.BlockSpec((1, N_CLUSTERS, PAD_D), lambda b: (b, 0, 0)),
        ],
        out_specs=[
            pl.BlockSpec((1, N_CLUSTERS, PAD_D), lambda b: (b, 0, 0)),
            pl.BlockSpec((1, 1, N_POINTS), lambda b: (b, 0, 0)),
            pl.BlockSpec((1, 1, N_POINTS), lambda b: (b, 0, 0)),
        ],
        out_shape=[
            jax.ShapeDtypeStruct((B, N_CLUSTERS, PAD_D), jnp.float32),
            jax.ShapeDtypeStruct((B, 1, N_POINTS), jnp.int32),
            jax.ShapeDtypeStruct((B, 1, N_POINTS), jnp.float32),
        ],
        compiler_params=pltpu.CompilerParams(
            dimension_semantics=("parallel",)),
    )(xp, xt, cp)

    KT = 4
    KB = N_POINTS // KT
    cf = pl.pallas_call(
        _feat_kernel,
        grid=(B, KT),
        in_specs=[
            pl.BlockSpec((1, 1, KB), lambda b, t: (b, 0, t)),
            pl.BlockSpec((1, 1, KB), lambda b, t: (b, 0, t)),
            pl.BlockSpec((1, F, KB), lambda b, t: (b, 0, t)),
        ],
        out_specs=pl.BlockSpec((1, F, N_CLUSTERS), lambda b, t: (b, 0, 0)),
        out_shape=jax.ShapeDtypeStruct((B, F, N_CLUSTERS), jnp.float32),
        compiler_params=pltpu.CompilerParams(
            dimension_semantics=("parallel", "arbitrary")),
    )(closest, v, features)

    cc = cents[:, :, :3]
    return cc, cf, closest[:, 0, :]
